# baseline (speedup 1.0000x reference)
import numpy as np
import jax
import jax.numpy as jnp
from jax.experimental import pallas as pl
from jax.experimental.pallas import tpu as pltpu

N_POS_DIMS = 3
N_LEVELS = 4
N_FEATURES = 4
LOG2_HASHMAP_SIZE = 10
BASE_RESOLUTION = 2
PER_LEVEL_SCALE = 2.0

PRIME_Y = -1640531535
PRIME_Z = 805459861

LEVEL_META = []
_pad_off = 0
for _lvl in range(N_LEVELS):
    _scale = float(PER_LEVEL_SCALE ** _lvl * BASE_RESOLUTION - 1.0)
    _res = int(-(-_scale // 1)) + 1
    _len = _res ** N_POS_DIMS
    _len = (_len + 7) // 8 * 8
    _len = min(_len, 1 << LOG2_HASHMAP_SIZE)
    _stride = 1
    for _d in range(N_POS_DIMS):
        _stride *= _res
        if _stride > _len:
            break
    _use_hash = _len < _stride
    assert (_len & (_len - 1)) == 0
    LEVEL_META.append((_scale, _res, _len, _use_hash, _pad_off, (_len + 127) // 128))
    _pad_off += ((_len + 127) // 128) * 128
T_TOTAL = _pad_off

_SLOT_INDEX = []
SLOT_BASE = []
for _lvl, (_s, _res, _hsize, _use_hash, _off, _nch) in enumerate(LEVEL_META):
    SLOT_BASE.append(len(_SLOT_INDEX))
    if _use_hash:
        for _k in range(_nch):
            _SLOT_INDEX.append([_off + _k * 128 + _l for _l in range(128)])
    else:
        for _c in range(8):
            _dx, _dy, _dz = (_c >> 2) & 1, (_c >> 1) & 1, _c & 1
            _coff = _dx + _dy * _res + _dz * _res * _res
            for _k in range(_nch):
                _SLOT_INDEX.append(
                    [_off + ((_k * 128 + _l + _coff) & (_hsize - 1))
                     if _hsize > 128 else
                     _off + ((_l + _coff) % _hsize)
                     for _l in range(128)])
N_SLOTS = len(_SLOT_INDEX)
_SLOT_INDEX_NP = np.asarray(_SLOT_INDEX, np.int32)

SUB = 128
BLOCK_ROWS = SUB * 8
BLOCK_PTS = BLOCK_ROWS * 128


def _hashgrid_kernel(coords_ref, tab_ref, out_ref):
    for s in range(SUB):
        r0 = s * 8
        x = coords_ref[0, r0:r0 + 8, :]
        y = coords_ref[1, r0:r0 + 8, :]
        z = coords_ref[2, r0:r0 + 8, :]

        for lvl in range(N_LEVELS):
            scale, res, hsize, use_hash, _, nch = LEVEL_META[lvl]
            sbase = SLOT_BASE[lvl]
            px = x * scale + 0.5
            py = y * scale + 0.5
            pz = z * scale + 0.5
            ix = jnp.floor(px).astype(jnp.int32)
            iy = jnp.floor(py).astype(jnp.int32)
            iz = jnp.floor(pz).astype(jnp.int32)
            fx = px - ix.astype(jnp.float32)
            fy = py - iy.astype(jnp.float32)
            fz = pz - iz.astype(jnp.float32)
            omx = 1.0 - fx
            omy = 1.0 - fy
            omz = 1.0 - fz

            wxy = ((omx * omy, omx * fy), (fx * omy, fx * fy))
            wz = (omz, fz)

            if use_hash:
                tx = (ix, ix + 1)
                ty = (iy * PRIME_Y, (iy + 1) * PRIME_Y)
                tz = (iz * PRIME_Z, (iz + 1) * PRIME_Z)
            else:
                base = ix + iy * res + iz * (res * res)
                if hsize <= 128:
                    lo = base & (hsize - 1)
                    masks = []
                else:
                    lo = base & 127
                    hi = (base >> 7) & (nch - 1)
                    masks = [hi == k for k in range(1, nch)]

            accs = [None] * N_FEATURES
            accs_b = [None] * N_FEATURES
            for c in range(8):
                dx, dy, dz = (c >> 2) & 1, (c >> 1) & 1, c & 1
                w = wxy[dx][dy] * wz[dz]
                if use_hash:
                    idx = (tx[dx] ^ ty[dy] ^ tz[dz]) & (hsize - 1)
                    c_lo = idx & 127
                    c_hi = idx >> 7
                    c_masks = [c_hi == k for k in range(1, nch)]
                    c_slot = sbase
                else:
                    c_lo = lo
                    c_masks = masks
                    c_slot = sbase + c * nch
                for p in range(2):
                    g = jnp.take_along_axis(tab_ref[c_slot, p], c_lo, axis=1)
                    for k in range(1, nch):
                        cand = jnp.take_along_axis(tab_ref[c_slot + k, p], c_lo,
                                                   axis=1)
                        g = jnp.where(c_masks[k - 1], cand, g)
                    f_even = pltpu.bitcast(g << 16, jnp.float32)
                    f_odd = pltpu.bitcast(g & jnp.int32(-65536), jnp.float32)
                    tgt = accs if c < 4 else accs_b
                    for f, val in ((2 * p, f_even), (2 * p + 1, f_odd)):
                        contrib = w * val
                        tgt[f] = contrib if tgt[f] is None else tgt[f] + contrib

            for f in range(N_FEATURES):
                out_ref[lvl * N_FEATURES + f, r0:r0 + 8, :] = accs[f] + accs_b[f]


def _pack_tables(table_t):
    bf = table_t[:N_FEATURES].astype(jnp.bfloat16)
    pairs = []
    for p in range(2):
        st = jnp.stack([bf[2 * p], bf[2 * p + 1]], axis=-1)
        pairs.append(jax.lax.bitcast_convert_type(st, jnp.int32))
    packed = jnp.stack(pairs)
    slots = packed[:, _SLOT_INDEX_NP]
    slots = slots.transpose(1, 0, 2)
    return jnp.broadcast_to(slots[:, :, None, :], (N_SLOTS, 2, 8, 128))


@jax.jit
def kernel(coords, table_t):
    n = coords.shape[0]
    n_pad = (n + BLOCK_PTS - 1) // BLOCK_PTS * BLOCK_PTS
    m = n_pad // 128
    c = jnp.pad(coords.astype(jnp.float32), ((0, n_pad - n), (0, 0)))
    c = c.T.reshape(N_POS_DIMS, m, 128)
    tab = _pack_tables(table_t.astype(jnp.float32))

    cost = pl.CostEstimate(
        flops=n_pad * N_LEVELS * 8 * (N_FEATURES * 4 + 8),
        transcendentals=0,
        bytes_accessed=n_pad * (N_POS_DIMS + N_LEVELS * N_FEATURES) * 4,
    )

    out = pl.pallas_call(
        _hashgrid_kernel,
        out_shape=jax.ShapeDtypeStruct((N_LEVELS * N_FEATURES, m, 128), jnp.float32),
        grid=(m // BLOCK_ROWS,),
        in_specs=[
            pl.BlockSpec((N_POS_DIMS, BLOCK_ROWS, 128), lambda i: (0, i, 0)),
            pl.BlockSpec((N_SLOTS, 2, 8, 128), lambda i: (0, 0, 0, 0)),
        ],
        out_specs=pl.BlockSpec((N_LEVELS * N_FEATURES, BLOCK_ROWS, 128),
                               lambda i: (0, i, 0)),
        compiler_params=pltpu.CompilerParams(dimension_semantics=("parallel",)),
        cost_estimate=cost,
    )(c, tab)

    return out.transpose(1, 2, 0).reshape(n_pad, N_LEVELS * N_FEATURES)[:n]


# speedup vs baseline: 1.0031x; 1.0031x over previous
import numpy as np
import jax
import jax.numpy as jnp
from jax.experimental import pallas as pl
from jax.experimental.pallas import tpu as pltpu

N_POS_DIMS = 3
N_LEVELS = 4
N_FEATURES = 4
LOG2_HASHMAP_SIZE = 10
BASE_RESOLUTION = 2
PER_LEVEL_SCALE = 2.0

PRIME_Y = -1640531535
PRIME_Z = 805459861

LEVEL_META = []
_pad_off = 0
for _lvl in range(N_LEVELS):
    _scale = float(PER_LEVEL_SCALE ** _lvl * BASE_RESOLUTION - 1.0)
    _res = int(-(-_scale // 1)) + 1
    _len = _res ** N_POS_DIMS
    _len = (_len + 7) // 8 * 8
    _len = min(_len, 1 << LOG2_HASHMAP_SIZE)
    _stride = 1
    for _d in range(N_POS_DIMS):
        _stride *= _res
        if _stride > _len:
            break
    _use_hash = _len < _stride
    assert (_len & (_len - 1)) == 0
    LEVEL_META.append((_scale, _res, _len, _use_hash, _pad_off, (_len + 127) // 128))
    _pad_off += ((_len + 127) // 128) * 128
T_TOTAL = _pad_off

_SLOT_INDEX = []
SLOT_BASE = []
for _lvl, (_s, _res, _hsize, _use_hash, _off, _nch) in enumerate(LEVEL_META):
    SLOT_BASE.append(len(_SLOT_INDEX))
    if _use_hash:
        for _k in range(_nch):
            _SLOT_INDEX.append([_off + _k * 128 + _l for _l in range(128)])
    else:
        for _c in range(8):
            _dx, _dy, _dz = (_c >> 2) & 1, (_c >> 1) & 1, _c & 1
            _coff = _dx + _dy * _res + _dz * _res * _res
            for _k in range(_nch):
                _SLOT_INDEX.append(
                    [_off + ((_k * 128 + _l + _coff) & (_hsize - 1))
                     if _hsize > 128 else
                     _off + ((_l + _coff) % _hsize)
                     for _l in range(128)])
N_SLOTS = len(_SLOT_INDEX)
_SLOT_INDEX_NP = np.asarray(_SLOT_INDEX, np.int32)

SUB = 64
BLOCK_ROWS = SUB * 8
BLOCK_PTS = BLOCK_ROWS * 128


def _hashgrid_kernel(coords_ref, tab_ref, out_ref):
    for s in range(SUB):
        r0 = s * 8
        x = coords_ref[0, r0:r0 + 8, :]
        y = coords_ref[1, r0:r0 + 8, :]
        z = coords_ref[2, r0:r0 + 8, :]

        for lvl in range(N_LEVELS):
            scale, res, hsize, use_hash, _, nch = LEVEL_META[lvl]
            sbase = SLOT_BASE[lvl]
            px = x * scale + 0.5
            py = y * scale + 0.5
            pz = z * scale + 0.5
            ix = jnp.floor(px).astype(jnp.int32)
            iy = jnp.floor(py).astype(jnp.int32)
            iz = jnp.floor(pz).astype(jnp.int32)
            fx = px - ix.astype(jnp.float32)
            fy = py - iy.astype(jnp.float32)
            fz = pz - iz.astype(jnp.float32)
            omx = 1.0 - fx
            omy = 1.0 - fy
            omz = 1.0 - fz

            wxy = ((omx * omy, omx * fy), (fx * omy, fx * fy))
            wz = (omz, fz)

            if use_hash:
                tx = (ix, ix + 1)
                ty = (iy * PRIME_Y, (iy + 1) * PRIME_Y)
                tz = (iz * PRIME_Z, (iz + 1) * PRIME_Z)
            else:
                base = ix + iy * res + iz * (res * res)
                if hsize <= 128:
                    lo = base & (hsize - 1)
                    masks = []
                else:
                    lo = base & 127
                    hi = (base >> 7) & (nch - 1)
                    masks = [hi == k for k in range(1, nch)]

            accs = [None] * N_FEATURES
            accs_b = [None] * N_FEATURES
            for c in range(8):
                dx, dy, dz = (c >> 2) & 1, (c >> 1) & 1, c & 1
                w = wxy[dx][dy] * wz[dz]
                if use_hash:
                    idx = (tx[dx] ^ ty[dy] ^ tz[dz]) & (hsize - 1)
                    c_lo = idx & 127
                    c_hi = idx >> 7
                    c_masks = [c_hi == k for k in range(1, nch)]
                    c_slot = sbase
                else:
                    c_lo = lo
                    c_masks = masks
                    c_slot = sbase + c * nch
                for p in range(2):
                    g = jnp.take_along_axis(tab_ref[c_slot, p], c_lo, axis=1)
                    for k in range(1, nch):
                        cand = jnp.take_along_axis(tab_ref[c_slot + k, p], c_lo,
                                                   axis=1)
                        g = jnp.where(c_masks[k - 1], cand, g)
                    f_even = pltpu.bitcast(g << 16, jnp.float32)
                    f_odd = pltpu.bitcast(g & jnp.int32(-65536), jnp.float32)
                    tgt = accs if c < 4 else accs_b
                    for f, val in ((2 * p, f_even), (2 * p + 1, f_odd)):
                        contrib = w * val
                        tgt[f] = contrib if tgt[f] is None else tgt[f] + contrib

            for f in range(N_FEATURES):
                out_ref[lvl * N_FEATURES + f, r0:r0 + 8, :] = accs[f] + accs_b[f]


def _pack_tables(table_t):
    bf = table_t[:N_FEATURES].astype(jnp.bfloat16)
    pairs = []
    for p in range(2):
        st = jnp.stack([bf[2 * p], bf[2 * p + 1]], axis=-1)
        pairs.append(jax.lax.bitcast_convert_type(st, jnp.int32))
    packed = jnp.stack(pairs)
    slots = packed[:, _SLOT_INDEX_NP]
    slots = slots.transpose(1, 0, 2)
    return jnp.broadcast_to(slots[:, :, None, :], (N_SLOTS, 2, 8, 128))


@jax.jit
def kernel(coords, table_t):
    n = coords.shape[0]
    n_pad = (n + BLOCK_PTS - 1) // BLOCK_PTS * BLOCK_PTS
    m = n_pad // 128
    c = jnp.pad(coords.astype(jnp.float32), ((0, n_pad - n), (0, 0)))
    c = c.T.reshape(N_POS_DIMS, m, 128)
    tab = _pack_tables(table_t.astype(jnp.float32))

    cost = pl.CostEstimate(
        flops=n_pad * N_LEVELS * 8 * (N_FEATURES * 4 + 8),
        transcendentals=0,
        bytes_accessed=n_pad * (N_POS_DIMS + N_LEVELS * N_FEATURES) * 4,
    )

    out = pl.pallas_call(
        _hashgrid_kernel,
        out_shape=jax.ShapeDtypeStruct((N_LEVELS * N_FEATURES, m, 128), jnp.float32),
        grid=(m // BLOCK_ROWS,),
        in_specs=[
            pl.BlockSpec((N_POS_DIMS, BLOCK_ROWS, 128), lambda i: (0, i, 0)),
            pl.BlockSpec((N_SLOTS, 2, 8, 128), lambda i: (0, 0, 0, 0)),
        ],
        out_specs=pl.BlockSpec((N_LEVELS * N_FEATURES, BLOCK_ROWS, 128),
                               lambda i: (0, i, 0)),
        compiler_params=pltpu.CompilerParams(dimension_semantics=("parallel",)),
        cost_estimate=cost,
    )(c, tab)

    return out.transpose(1, 2, 0).reshape(n_pad, N_LEVELS * N_FEATURES)[:n]
